# baseline (speedup 1.0000x reference)
import functools

import jax
import jax.numpy as jnp
from jax import lax
from jax.experimental import pallas as pl
from jax.experimental.pallas import tpu as pltpu
from jax.experimental.pallas import tpu_sc as plsc

NUM_FIELDS = 26
VOCAB = 100000
DIM = 32
BATCH = 1024
HIST = 50

N_ROWS = BATCH * HIST * NUM_FIELDS
NUM_CORES = 2
NUM_SUBCORES = 16
NW = NUM_CORES * NUM_SUBCORES
PER_W = N_ROWS // NW
LANES = 16
BLK = 128
BLOCKS_PER_W = PER_W // BLK
NBUF = 13
OUTER = BLOCKS_PER_W // NBUF

_MESH = plsc.VectorSubcoreMesh(core_axis_name="c", subcore_axis_name="s")


@functools.partial(
    pl.kernel,
    mesh=_MESH,
    out_type=jax.ShapeDtypeStruct((N_ROWS, DIM), jnp.float32),
    scratch_types=[
        pltpu.VMEM((BLOCKS_PER_W, BLK), jnp.int32),
        pltpu.VMEM((NBUF, BLK), jnp.int32),
        pltpu.VMEM((NBUF, BLK, DIM), jnp.float32),
        pltpu.SemaphoreType.DMA,
        pltpu.SemaphoreType.DMA,
    ],
    compiler_params=pltpu.CompilerParams(use_tc_tiling_on_sc=False),
)
def _gather_kernel(table_hbm, idx_hbm, out_hbm, idx_v, offs_v, rows_v, sem_g, sem_o):
    wid = lax.axis_index("s") * NUM_CORES + lax.axis_index("c")
    wblock0 = wid * BLOCKS_PER_W

    pltpu.sync_copy(idx_hbm.at[pl.ds(wblock0, BLOCKS_PER_W)], idx_v)

    for b in range(NBUF):
        for k in range(BLK // LANES):
            sl = pl.ds(k * LANES, LANES)
            lane_n = b * BLK + k * LANES + lax.iota(jnp.int32, LANES)
            offs_v[b, sl] = (lane_n % NUM_FIELDS) * VOCAB

    def outer(i, carry):
        gathers = []
        for b in range(NBUF):
            j = i * NBUF + b
            for k in range(BLK // LANES):
                sl = pl.ds(k * LANES, LANES)
                idx_v[j, sl] = idx_v[j, sl] + offs_v[b, sl]
            gathers.append(
                pltpu.async_copy(table_hbm.at[idx_v.at[j]], rows_v.at[b], sem_g))
        stores = []
        for b in range(NBUF):
            j = i * NBUF + b
            gathers[b].wait()
            base = (wblock0 + j) * BLK
            stores.append(
                pltpu.async_copy(rows_v.at[b], out_hbm.at[pl.ds(base, BLK)], sem_o))
        for h in stores:
            h.wait()
        return carry

    lax.fori_loop(0, OUTER, outer, 0)


def kernel(x, tables):
    flat_idx = x.reshape(N_ROWS // BLK, BLK)
    flat_tab = tables.reshape(NUM_FIELDS * VOCAB, DIM)
    out = _gather_kernel(flat_tab, flat_idx)
    return out.reshape(BATCH, HIST, NUM_FIELDS * DIM)


# speedup vs baseline: 1.0015x; 1.0015x over previous
import functools

import jax
import jax.numpy as jnp
from jax import lax
from jax.experimental import pallas as pl
from jax.experimental.pallas import tpu as pltpu
from jax.experimental.pallas import tpu_sc as plsc

NUM_FIELDS = 26
VOCAB = 100000
DIM = 32
BATCH = 1024
HIST = 50

N_ROWS = BATCH * HIST * NUM_FIELDS
NUM_CORES = 2
NUM_SUBCORES = 16
NW = NUM_CORES * NUM_SUBCORES
PER_W = N_ROWS // NW
LANES = 16
BLK = 128
BLOCKS_PER_W = PER_W // BLK
NBUF = 13
OUTER = BLOCKS_PER_W // NBUF

_MESH = plsc.VectorSubcoreMesh(core_axis_name="c", subcore_axis_name="s")


@functools.partial(
    pl.kernel,
    mesh=_MESH,
    out_type=jax.ShapeDtypeStruct((N_ROWS, DIM), jnp.float32),
    scratch_types=[
        pltpu.VMEM((BLOCKS_PER_W, BLK), jnp.int32),
        pltpu.VMEM((NBUF, BLK), jnp.int32),
        pltpu.VMEM((NBUF, BLK, DIM), jnp.float32),
        pltpu.SemaphoreType.DMA,
        pltpu.SemaphoreType.DMA,
    ],
    compiler_params=pltpu.CompilerParams(use_tc_tiling_on_sc=False),
)
def _gather_kernel(table_hbm, idx_hbm, out_hbm, idx_v, offs_v, rows_v, sem_g, sem_o):
    wid = lax.axis_index("s") * NUM_CORES + lax.axis_index("c")
    wblock0 = wid * BLOCKS_PER_W

    pltpu.sync_copy(idx_hbm.at[pl.ds(wblock0, BLOCKS_PER_W)], idx_v)

    for b in range(NBUF):
        for k in range(BLK // LANES):
            sl = pl.ds(k * LANES, LANES)
            lane_n = b * BLK + k * LANES + lax.iota(jnp.int32, LANES)
            offs_v[b, sl] = (lane_n % NUM_FIELDS) * VOCAB

    def outer(i, carry):
        gathers = []
        for b in range(NBUF):
            j = i * NBUF + b

            @pl.when(i > 0)
            def _drain():
                pltpu.make_async_copy(
                    out_hbm.at[pl.ds(0, BLK)], rows_v.at[b], sem_o).wait()

            for k in range(BLK // LANES):
                sl = pl.ds(k * LANES, LANES)
                idx_v[j, sl] = idx_v[j, sl] + offs_v[b, sl]
            gathers.append(
                pltpu.async_copy(table_hbm.at[idx_v.at[j]], rows_v.at[b], sem_g))
        for b in range(NBUF):
            j = i * NBUF + b
            gathers[b].wait()
            base = (wblock0 + j) * BLK
            pltpu.async_copy(rows_v.at[b], out_hbm.at[pl.ds(base, BLK)], sem_o)
        return carry

    lax.fori_loop(0, OUTER, outer, 0)
    for b in range(NBUF):
        pltpu.make_async_copy(
            out_hbm.at[pl.ds(0, BLK)], rows_v.at[b], sem_o).wait()


def kernel(x, tables):
    flat_idx = x.reshape(N_ROWS // BLK, BLK)
    flat_tab = tables.reshape(NUM_FIELDS * VOCAB, DIM)
    out = _gather_kernel(flat_tab, flat_idx)
    return out.reshape(BATCH, HIST, NUM_FIELDS * DIM)
